# baseline (speedup 1.0000x reference)
import functools

import jax
import jax.numpy as jnp
from jax import lax
from jax.experimental import pallas as pl
from jax.experimental.pallas import tpu as pltpu
from jax.experimental.pallas import tpu_sc as plsc

VOCAB = 30522
HIDDEN = 768
BATCH = 1024
SEQ = 200
L = 16

NC = 2
NS = 16
NW = NC * NS
RPW = BATCH // NW
SPW = RPW * SEQ
CHUNK = 128
NCHUNK = SPW // CHUNK


def _proj_body(e_ref, w_ref, o_ref):
    x = jnp.dot(e_ref[...], w_ref[...], preferred_element_type=jnp.float32)

    def rne_hi(u):
        return (u + 0x7FFF + ((u >> 16) & 1)) & jnp.uint32(0xFFFF0000)

    u0 = lax.bitcast_convert_type(x[:, 0], jnp.uint32)
    u1 = lax.bitcast_convert_type(x[:, 1], jnp.uint32)
    o_ref[...] = (rne_hi(u0) >> 16) | rne_hi(u1)


def _project_table(emb, w):
    bm = 1024
    return pl.pallas_call(
        _proj_body,
        grid=(pl.cdiv(VOCAB, bm),),
        in_specs=[
            pl.BlockSpec((bm, HIDDEN), lambda i: (i, 0)),
            pl.BlockSpec((HIDDEN, 2), lambda i: (0, 0)),
        ],
        out_specs=pl.BlockSpec((bm,), lambda i: (i,)),
        out_shape=jax.ShapeDtypeStruct((VOCAB,), jnp.uint32),
    )(emb, w)


def _sc_pool_body(ids_hbm, mask_hbm, p_hbm, b_hbm, out_hbm,
                  idx_v, mask_v, g_v, b_v, out_v, sem):
    wid = lax.axis_index("s") * NC + lax.axis_index("c")

    pltpu.sync_copy(ids_hbm.at[pl.ds(wid * SPW, SPW)], idx_v)
    pltpu.sync_copy(mask_hbm.at[pl.ds(wid * SPW, SPW)], mask_v)
    pltpu.sync_copy(b_hbm, b_v)

    wave = 25
    for g in range(0, NCHUNK, wave):
        handles = [
            pltpu.async_copy(
                p_hbm.at[idx_v.at[pl.ds(j * CHUNK, CHUNK)]],
                g_v.at[pl.ds(j * CHUNK, CHUNK)],
                sem,
            )
            for j in range(g, g + wave)
        ]
        for h in handles:
            h.wait()

    iota = lax.iota(jnp.int32, L)
    bvec = b_v[...]

    dnums = lax.GatherDimensionNumbers(
        offset_dims=(), collapsed_slice_dims=(0,), start_index_map=(0,))

    def shuffle(x, perm):
        return lax.gather(x, perm[:, None], dnums, (1,),
                          mode=lax.GatherScatterMode.PROMISE_IN_BOUNDS)

    def lane_sum(x):
        for k in (8, 4, 2, 1):
            x = x + shuffle(x, iota ^ k)
        return x

    def unpack(g):
        c0 = lax.bitcast_convert_type(g << 16, jnp.float32)
        c1 = lax.bitcast_convert_type(g & jnp.uint32(0xFFFF0000),
                                      jnp.float32)
        return c0, c1

    zero = jnp.zeros((L,), jnp.float32)
    bpair = shuffle(bvec, iota & 1)

    def row_sums(r):
        s_base = r * SEQ

        def chunk(j, carry):
            a0, a1 = carry
            s0 = s_base + j * L
            m = mask_v[pl.ds(s0, L)]
            c0, c1 = unpack(g_v[pl.ds(s0, L)])
            return a0 + c0 * m, a1 + c1 * m

        acc0, acc1 = lax.fori_loop(0, SEQ // L, chunk, (zero, zero))

        s0 = s_base + SEQ - L
        m = mask_v[pl.ds(s0, L)]
        m = jnp.where(iota < (L - SEQ % L), 0.0, m)
        c0, c1 = unpack(g_v[pl.ds(s0, L)])
        acc0 = acc0 + c0 * m
        acc1 = acc1 + c1 * m
        return lane_sum(acc0), lane_sum(acc1)

    def group_body(gi, _):
        vacc = zero
        for q in range(8):
            t0, t1 = row_sums(gi * 8 + q)
            vacc = jnp.where(iota == 2 * q, t0, vacc)
            vacc = jnp.where(iota == 2 * q + 1, t1, vacc)
        out_v[pl.ds(gi * L, L)] = vacc / float(SEQ) + bpair
        return 0

    lax.fori_loop(0, RPW // 8, group_body, 0)
    pltpu.sync_copy(out_v, out_hbm.at[pl.ds(wid * 2 * RPW, 2 * RPW)])


@functools.cache
def _make_sc_pool():
    @functools.partial(
        pl.kernel,
        mesh=plsc.VectorSubcoreMesh(core_axis_name="c", subcore_axis_name="s"),
        out_type=jax.ShapeDtypeStruct((BATCH * 2,), jnp.float32),
        scratch_types=[
            pltpu.VMEM((SPW,), jnp.int32),
            pltpu.VMEM((SPW,), jnp.float32),
            pltpu.VMEM((SPW,), jnp.uint32),
            pltpu.VMEM((L,), jnp.float32),
            pltpu.VMEM((2 * RPW,), jnp.float32),
            pltpu.SemaphoreType.DMA,
        ],
    )
    def _sc_pool(ids_hbm, mask_hbm, p_hbm, b_hbm, out_hbm, *scratch):
        _sc_pool_body(ids_hbm, mask_hbm, p_hbm, b_hbm, out_hbm, *scratch)

    return _sc_pool


def kernel(input_ids, attention_mask, embedding_table, classifier_w,
           classifier_b):
    p = _project_table(embedding_table, classifier_w.astype(jnp.float32))
    ids = input_ids.astype(jnp.int32).reshape(BATCH * SEQ)
    mask = attention_mask.astype(jnp.float32).reshape(BATCH * SEQ)
    b16 = jnp.pad(classifier_b.astype(jnp.float32), (0, L - 2))
    return _make_sc_pool()(ids, mask, p, b16).reshape(BATCH, 2)


# speedup vs baseline: 1.0271x; 1.0271x over previous
import functools

import jax
import jax.numpy as jnp
from jax import lax
from jax.experimental import pallas as pl
from jax.experimental.pallas import tpu as pltpu
from jax.experimental.pallas import tpu_sc as plsc

VOCAB = 30522
HIDDEN = 768
BATCH = 1024
SEQ = 200
L = 16

NC = 2
NS = 16
NW = NC * NS
RPW = BATCH // NW
SPLIT = 128
REST = SEQ - SPLIT


def _proj_body(e_ref, w_ref, o_ref):
    x = jnp.dot(e_ref[...], w_ref[...], preferred_element_type=jnp.float32)

    def rne_hi(u):
        return (u + 0x7FFF + ((u >> 16) & 1)) & jnp.uint32(0xFFFF0000)

    u0 = lax.bitcast_convert_type(x[:, 0], jnp.uint32)
    u1 = lax.bitcast_convert_type(x[:, 1], jnp.uint32)
    o_ref[...] = (rne_hi(u0) >> 16) | rne_hi(u1)


def _project_table(emb, w):
    bm = 1024
    return pl.pallas_call(
        _proj_body,
        grid=(pl.cdiv(VOCAB, bm),),
        in_specs=[
            pl.BlockSpec((bm, HIDDEN), lambda i: (i, 0)),
            pl.BlockSpec((HIDDEN, 2), lambda i: (0, 0)),
        ],
        out_specs=pl.BlockSpec((bm,), lambda i: (i,)),
        out_shape=jax.ShapeDtypeStruct((VOCAB,), jnp.uint32),
    )(emb, w)


def _sc_pool_body(ids_hbm, mask_hbm, p_hbm, b_hbm, out_hbm,
                  idx_v, mask_v, g_v, b_v, out_v, sem_a, sem_b):
    wid = lax.axis_index("s") * NC + lax.axis_index("c")
    base = wid * RPW

    pltpu.sync_copy(ids_hbm.at[pl.ds(base, RPW), :], idx_v)

    def fire(r, sem):
        return [
            pltpu.async_copy(p_hbm.at[idx_v.at[r, pl.ds(0, SPLIT)]],
                             g_v.at[r, pl.ds(0, SPLIT)], sem),
            pltpu.async_copy(p_hbm.at[idx_v.at[r, pl.ds(SPLIT, REST)]],
                             g_v.at[r, pl.ds(SPLIT, REST)], sem),
        ]

    handles_a, handles_b = [], []
    for r in range(RPW // 2):
        handles_a += fire(r, sem_a)
    for r in range(RPW // 2, RPW):
        handles_b += fire(r, sem_b)

    pltpu.sync_copy(mask_hbm.at[pl.ds(base, RPW), :], mask_v)
    pltpu.sync_copy(b_hbm, b_v)

    iota = lax.iota(jnp.int32, L)
    bvec = b_v[...]

    dnums = lax.GatherDimensionNumbers(
        offset_dims=(), collapsed_slice_dims=(0,), start_index_map=(0,))

    def shuffle(x, perm):
        return lax.gather(x, perm[:, None], dnums, (1,),
                          mode=lax.GatherScatterMode.PROMISE_IN_BOUNDS)

    def lane_sum(x):
        for k in (8, 4, 2, 1):
            x = x + shuffle(x, iota ^ k)
        return x

    def unpack(g):
        c0 = lax.bitcast_convert_type(g << 16, jnp.float32)
        c1 = lax.bitcast_convert_type(g & jnp.uint32(0xFFFF0000),
                                      jnp.float32)
        return c0, c1

    zero = jnp.zeros((L,), jnp.float32)
    bpair = shuffle(bvec, iota & 1)

    def row_sums(r):
        def chunk(j, carry):
            a0, a1 = carry
            s0 = j * L
            m = mask_v[r, pl.ds(s0, L)]
            c0, c1 = unpack(g_v[r, pl.ds(s0, L)])
            return a0 + c0 * m, a1 + c1 * m

        acc0, acc1 = lax.fori_loop(0, SEQ // L, chunk, (zero, zero))

        s0 = SEQ - L
        m = mask_v[r, pl.ds(s0, L)]
        m = jnp.where(iota < (L - SEQ % L), 0.0, m)
        c0, c1 = unpack(g_v[r, pl.ds(s0, L)])
        acc0 = acc0 + c0 * m
        acc1 = acc1 + c1 * m
        return lane_sum(acc0), lane_sum(acc1)

    def group_body(gi, _):
        vacc = zero
        for q in range(8):
            t0, t1 = row_sums(gi * 8 + q)
            vacc = jnp.where(iota == 2 * q, t0, vacc)
            vacc = jnp.where(iota == 2 * q + 1, t1, vacc)
        out_v[pl.ds(gi * L, L)] = vacc / float(SEQ) + bpair
        return 0

    for h in handles_a:
        h.wait()
    lax.fori_loop(0, 2, group_body, 0)
    for h in handles_b:
        h.wait()
    lax.fori_loop(2, 4, group_body, 0)

    pltpu.sync_copy(out_v, out_hbm.at[pl.ds(wid * 2 * RPW, 2 * RPW)])


@functools.cache
def _make_sc_pool():
    @functools.partial(
        pl.kernel,
        mesh=plsc.VectorSubcoreMesh(core_axis_name="c", subcore_axis_name="s"),
        out_type=jax.ShapeDtypeStruct((BATCH * 2,), jnp.float32),
        scratch_types=[
            pltpu.VMEM((RPW, SEQ), jnp.int32),
            pltpu.VMEM((RPW, SEQ), jnp.float32),
            pltpu.VMEM((RPW, SEQ), jnp.uint32),
            pltpu.VMEM((L,), jnp.float32),
            pltpu.VMEM((2 * RPW,), jnp.float32),
            pltpu.SemaphoreType.DMA,
            pltpu.SemaphoreType.DMA,
        ],
    )
    def _sc_pool(ids_hbm, mask_hbm, p_hbm, b_hbm, out_hbm, *scratch):
        _sc_pool_body(ids_hbm, mask_hbm, p_hbm, b_hbm, out_hbm, *scratch)

    return _sc_pool


def kernel(input_ids, attention_mask, embedding_table, classifier_w,
           classifier_b):
    p = _project_table(embedding_table, classifier_w.astype(jnp.float32))
    ids = input_ids.astype(jnp.int32)
    mask = attention_mask.astype(jnp.float32)
    b16 = jnp.pad(classifier_b.astype(jnp.float32), (0, L - 2))
    return _make_sc_pool()(ids, mask, p, b16).reshape(BATCH, 2)


# speedup vs baseline: 1.1422x; 1.1121x over previous
import functools

import jax
import jax.numpy as jnp
from jax import lax
from jax.experimental import pallas as pl
from jax.experimental.pallas import tpu as pltpu
from jax.experimental.pallas import tpu_sc as plsc

VOCAB = 30522
HIDDEN = 768
BATCH = 1024
SEQ = 200
L = 16

NC = 2
NS = 16
NW = NC * NS
RPW = BATCH // NW
SPLIT = 128
REST = SEQ - SPLIT


def _proj_body(e_ref, w_ref, o_ref):
    x = jnp.dot(e_ref[...], w_ref[...], preferred_element_type=jnp.float32)

    def rne_hi(u):
        return (u + 0x7FFF + ((u >> 16) & 1)) & jnp.uint32(0xFFFF0000)

    u0 = lax.bitcast_convert_type(x[:, 0], jnp.uint32)
    u1 = lax.bitcast_convert_type(x[:, 1], jnp.uint32)
    o_ref[...] = (rne_hi(u0) >> 16) | rne_hi(u1)


def _project_table(emb, w):
    bm = 2048
    return pl.pallas_call(
        _proj_body,
        grid=(pl.cdiv(VOCAB, bm),),
        in_specs=[
            pl.BlockSpec((bm, HIDDEN), lambda i: (i, 0)),
            pl.BlockSpec((HIDDEN, 2), lambda i: (0, 0)),
        ],
        out_specs=pl.BlockSpec((bm,), lambda i: (i,)),
        out_shape=jax.ShapeDtypeStruct((VOCAB,), jnp.uint32),
    )(emb, w)


def _sc_pool_body(ids_hbm, mask_hbm, p_hbm, b_hbm, out_hbm,
                  idx_v, mask_v, g_v, b_v, out_v, sem_a, sem_b):
    wid = lax.axis_index("s") * NC + lax.axis_index("c")
    base = wid * RPW

    pltpu.sync_copy(ids_hbm.at[pl.ds(base, RPW), :], idx_v)

    def fire(r, sem):
        return [
            pltpu.async_copy(p_hbm.at[idx_v.at[r, pl.ds(0, SPLIT)]],
                             g_v.at[r, pl.ds(0, SPLIT)], sem),
            pltpu.async_copy(p_hbm.at[idx_v.at[r, pl.ds(SPLIT, REST)]],
                             g_v.at[r, pl.ds(SPLIT, REST)], sem),
        ]

    handles_a, handles_b = [], []
    for r in range(RPW // 2):
        handles_a += fire(r, sem_a)
    for r in range(RPW // 2, RPW):
        handles_b += fire(r, sem_b)

    pltpu.sync_copy(mask_hbm.at[pl.ds(base, RPW), :], mask_v)
    pltpu.sync_copy(b_hbm, b_v)

    iota = lax.iota(jnp.int32, L)
    bvec = b_v[...]

    dnums = lax.GatherDimensionNumbers(
        offset_dims=(), collapsed_slice_dims=(0,), start_index_map=(0,))

    def shuffle(x, perm):
        return lax.gather(x, perm[:, None], dnums, (1,),
                          mode=lax.GatherScatterMode.PROMISE_IN_BOUNDS)

    def lane_sum(x):
        for k in (8, 4, 2, 1):
            x = x + shuffle(x, iota ^ k)
        return x

    def unpack(g):
        c0 = lax.bitcast_convert_type(g << 16, jnp.float32)
        c1 = lax.bitcast_convert_type(g & jnp.uint32(0xFFFF0000),
                                      jnp.float32)
        return c0, c1

    zero = jnp.zeros((L,), jnp.float32)
    bpair = shuffle(bvec, iota & 1)

    def row_sums(r):
        def chunk(j, carry):
            a0, a1 = carry
            s0 = j * L
            m = mask_v[r, pl.ds(s0, L)]
            c0, c1 = unpack(g_v[r, pl.ds(s0, L)])
            return a0 + c0 * m, a1 + c1 * m

        acc0, acc1 = lax.fori_loop(0, SEQ // L, chunk, (zero, zero))

        s0 = SEQ - L
        m = mask_v[r, pl.ds(s0, L)]
        m = jnp.where(iota < (L - SEQ % L), 0.0, m)
        c0, c1 = unpack(g_v[r, pl.ds(s0, L)])
        acc0 = acc0 + c0 * m
        acc1 = acc1 + c1 * m
        return lane_sum(acc0), lane_sum(acc1)

    def group_body(gi, _):
        vacc = zero
        for q in range(8):
            t0, t1 = row_sums(gi * 8 + q)
            vacc = jnp.where(iota == 2 * q, t0, vacc)
            vacc = jnp.where(iota == 2 * q + 1, t1, vacc)
        out_v[pl.ds(gi * L, L)] = vacc / float(SEQ) + bpair
        return 0

    for h in handles_a:
        h.wait()
    lax.fori_loop(0, 2, group_body, 0)
    for h in handles_b:
        h.wait()
    lax.fori_loop(2, 4, group_body, 0)

    pltpu.sync_copy(out_v, out_hbm.at[pl.ds(wid * 2 * RPW, 2 * RPW)])


@functools.cache
def _make_sc_pool():
    @functools.partial(
        pl.kernel,
        mesh=plsc.VectorSubcoreMesh(core_axis_name="c", subcore_axis_name="s"),
        out_type=jax.ShapeDtypeStruct((BATCH * 2,), jnp.float32),
        scratch_types=[
            pltpu.VMEM((RPW, SEQ), jnp.int32),
            pltpu.VMEM((RPW, SEQ), jnp.float32),
            pltpu.VMEM((RPW, SEQ), jnp.uint32),
            pltpu.VMEM((L,), jnp.float32),
            pltpu.VMEM((2 * RPW,), jnp.float32),
            pltpu.SemaphoreType.DMA,
            pltpu.SemaphoreType.DMA,
        ],
    )
    def _sc_pool(ids_hbm, mask_hbm, p_hbm, b_hbm, out_hbm, *scratch):
        _sc_pool_body(ids_hbm, mask_hbm, p_hbm, b_hbm, out_hbm, *scratch)

    return _sc_pool


def kernel(input_ids, attention_mask, embedding_table, classifier_w,
           classifier_b):
    p = _project_table(embedding_table, classifier_w.astype(jnp.float32))
    ids = input_ids.astype(jnp.int32)
    mask = attention_mask.astype(jnp.float32)
    b16 = jnp.pad(classifier_b.astype(jnp.float32), (0, L - 2))
    return _make_sc_pool()(ids, mask, p, b16).reshape(BATCH, 2)


# speedup vs baseline: 1.1854x; 1.0378x over previous
import functools

import jax
import jax.numpy as jnp
from jax import lax
from jax.experimental import pallas as pl
from jax.experimental.pallas import tpu as pltpu
from jax.experimental.pallas import tpu_sc as plsc

VOCAB = 30522
HIDDEN = 768
BATCH = 1024
SEQ = 200
L = 16

NC = 2
NS = 16
NW = NC * NS
RPW = BATCH // NW
SPLIT = 128
REST = SEQ - SPLIT


def _proj_body(e_ref, w_ref, o_ref):
    x = jnp.dot(e_ref[...], w_ref[...], preferred_element_type=jnp.float32)

    def rne_hi(u):
        return (u + 0x7FFF + ((u >> 16) & 1)) & jnp.uint32(0xFFFF0000)

    u0 = lax.bitcast_convert_type(x[:, 0], jnp.uint32)
    u1 = lax.bitcast_convert_type(x[:, 1], jnp.uint32)
    o_ref[...] = (rne_hi(u0) >> 16) | rne_hi(u1)


def _project_table(emb, w):
    bm = 3072
    return pl.pallas_call(
        _proj_body,
        grid=(pl.cdiv(VOCAB, bm),),
        in_specs=[
            pl.BlockSpec((bm, HIDDEN), lambda i: (i, 0)),
            pl.BlockSpec((HIDDEN, 2), lambda i: (0, 0)),
        ],
        out_specs=pl.BlockSpec((bm,), lambda i: (i,)),
        out_shape=jax.ShapeDtypeStruct((VOCAB,), jnp.uint32),
    )(emb, w)


def _sc_pool_body(ids_hbm, mask_hbm, p_hbm, b_hbm, out_hbm,
                  idx_v, mask_v, g_v, b_v, out_v, sem_a, sem_b):
    wid = lax.axis_index("s") * NC + lax.axis_index("c")
    base = wid * RPW

    pltpu.sync_copy(ids_hbm.at[pl.ds(base, RPW), :], idx_v)

    def fire(r, sem):
        return [
            pltpu.async_copy(p_hbm.at[idx_v.at[r, pl.ds(0, SPLIT)]],
                             g_v.at[r, pl.ds(0, SPLIT)], sem),
            pltpu.async_copy(p_hbm.at[idx_v.at[r, pl.ds(SPLIT, REST)]],
                             g_v.at[r, pl.ds(SPLIT, REST)], sem),
        ]

    handles_a, handles_b = [], []
    for r in range(RPW // 2):
        handles_a += fire(r, sem_a)
    for r in range(RPW // 2, RPW):
        handles_b += fire(r, sem_b)

    pltpu.sync_copy(mask_hbm.at[pl.ds(base, RPW), :], mask_v)
    pltpu.sync_copy(b_hbm, b_v)

    iota = lax.iota(jnp.int32, L)
    bvec = b_v[...]

    dnums = lax.GatherDimensionNumbers(
        offset_dims=(), collapsed_slice_dims=(0,), start_index_map=(0,))

    def shuffle(x, perm):
        return lax.gather(x, perm[:, None], dnums, (1,),
                          mode=lax.GatherScatterMode.PROMISE_IN_BOUNDS)

    def lane_sum(x):
        for k in (8, 4, 2, 1):
            x = x + shuffle(x, iota ^ k)
        return x

    def unpack(g):
        c0 = lax.bitcast_convert_type(g << 16, jnp.float32)
        c1 = lax.bitcast_convert_type(g & jnp.uint32(0xFFFF0000),
                                      jnp.float32)
        return c0, c1

    zero = jnp.zeros((L,), jnp.float32)
    bpair = shuffle(bvec, iota & 1)

    def row_sums(r):
        def chunk(j, carry):
            a0, a1 = carry
            s0 = j * L
            m = mask_v[r, pl.ds(s0, L)]
            c0, c1 = unpack(g_v[r, pl.ds(s0, L)])
            return a0 + c0 * m, a1 + c1 * m

        acc0, acc1 = lax.fori_loop(0, SEQ // L, chunk, (zero, zero))

        s0 = SEQ - L
        m = mask_v[r, pl.ds(s0, L)]
        m = jnp.where(iota < (L - SEQ % L), 0.0, m)
        c0, c1 = unpack(g_v[r, pl.ds(s0, L)])
        acc0 = acc0 + c0 * m
        acc1 = acc1 + c1 * m
        return lane_sum(acc0), lane_sum(acc1)

    def group_body(gi, _):
        vacc = zero
        for q in range(8):
            t0, t1 = row_sums(gi * 8 + q)
            vacc = jnp.where(iota == 2 * q, t0, vacc)
            vacc = jnp.where(iota == 2 * q + 1, t1, vacc)
        out_v[pl.ds(gi * L, L)] = vacc / float(SEQ) + bpair
        return 0

    for h in handles_a:
        h.wait()
    lax.fori_loop(0, 2, group_body, 0)
    for h in handles_b:
        h.wait()
    lax.fori_loop(2, 4, group_body, 0)

    pltpu.sync_copy(out_v, out_hbm.at[pl.ds(wid * 2 * RPW, 2 * RPW)])


@functools.cache
def _make_sc_pool():
    @functools.partial(
        pl.kernel,
        mesh=plsc.VectorSubcoreMesh(core_axis_name="c", subcore_axis_name="s"),
        out_type=jax.ShapeDtypeStruct((BATCH * 2,), jnp.float32),
        scratch_types=[
            pltpu.VMEM((RPW, SEQ), jnp.int32),
            pltpu.VMEM((RPW, SEQ), jnp.float32),
            pltpu.VMEM((RPW, SEQ), jnp.uint32),
            pltpu.VMEM((L,), jnp.float32),
            pltpu.VMEM((2 * RPW,), jnp.float32),
            pltpu.SemaphoreType.DMA,
            pltpu.SemaphoreType.DMA,
        ],
    )
    def _sc_pool(ids_hbm, mask_hbm, p_hbm, b_hbm, out_hbm, *scratch):
        _sc_pool_body(ids_hbm, mask_hbm, p_hbm, b_hbm, out_hbm, *scratch)

    return _sc_pool


def kernel(input_ids, attention_mask, embedding_table, classifier_w,
           classifier_b):
    p = _project_table(embedding_table, classifier_w.astype(jnp.float32))
    ids = input_ids.astype(jnp.int32)
    mask = attention_mask.astype(jnp.float32)
    b16 = jnp.pad(classifier_b.astype(jnp.float32), (0, L - 2))
    return _make_sc_pool()(ids, mask, p, b16).reshape(BATCH, 2)
